# Initial kernel scaffold; baseline (speedup 1.0000x reference)
import functools

import jax
import jax.numpy as jnp
from jax import lax
from jax.experimental import pallas as pl
from jax.experimental.pallas import tpu as pltpu
from jax.experimental.pallas import tpu_sc as plsc

N = 16384
L = 16
NC = 2
NS = 16
NW = NC * NS
CHUNK = N // NW


def _sc_body(src_hbm, tgt_hbm, embT_hbm, w_hbm, b_hbm, out_hbm,
             src_v, tgt_v, out_v, embT_v, w_v, b_v, v_ref):
    wid = lax.axis_index("s") * NC + lax.axis_index("c")
    base = wid * CHUNK

    pltpu.sync_copy(src_hbm.at[pl.ds(base, CHUNK)], src_v)
    pltpu.sync_copy(tgt_hbm.at[pl.ds(base, CHUNK)], tgt_v)
    pltpu.sync_copy(embT_hbm, embT_v)
    pltpu.sync_copy(w_hbm, w_v)
    pltpu.sync_copy(b_hbm, b_v)

    v_acc = jnp.zeros((L,), jnp.float32)
    for j in range(L):
        wj = plsc.load_gather(w_v, [jnp.full((L,), j, jnp.int32)])
        v_acc = v_acc + embT_v[j] * wj
    v_ref[...] = v_acc
    b_vec = b_v[...]

    for i in range(CHUNK // L):
        s_idx = src_v[pl.ds(i * L, L)]
        t_idx = tgt_v[pl.ds(i * L, L)]
        vs = plsc.load_gather(v_ref, [s_idx])
        vt = plsc.load_gather(v_ref, [t_idx])
        out_v[pl.ds(i * L, L)] = vs + vt + b_vec

    pltpu.sync_copy(out_v, out_hbm.at[pl.ds(base, CHUNK)])


def kernel(source, target, emb, W, b):
    embT_pad = jnp.zeros((L, L), jnp.float32).at[:, : emb.shape[0]].set(emb.T)
    w_flat = W.reshape(L)
    b_pad = jnp.broadcast_to(b.astype(jnp.float32), (L,))

    mesh = plsc.VectorSubcoreMesh(core_axis_name="c", subcore_axis_name="s")
    k = functools.partial(
        pl.kernel,
        mesh=mesh,
        out_type=jax.ShapeDtypeStruct((N,), jnp.float32),
        scratch_types=[
            pltpu.VMEM((CHUNK,), jnp.int32),
            pltpu.VMEM((CHUNK,), jnp.int32),
            pltpu.VMEM((CHUNK,), jnp.float32),
            pltpu.VMEM((L, L), jnp.float32),
            pltpu.VMEM((L,), jnp.float32),
            pltpu.VMEM((L,), jnp.float32),
            pltpu.VMEM((L,), jnp.float32),
        ],
    )(_sc_body)
    out = k(source.astype(jnp.int32), target.astype(jnp.int32),
            embT_pad, w_flat, b_pad)
    return out.reshape(N, 1)


# speedup vs baseline: 4.9883x; 4.9883x over previous
import functools

import jax
import jax.numpy as jnp
from jax import lax
from jax.experimental import pallas as pl
from jax.experimental.pallas import tpu as pltpu
from jax.experimental.pallas import tpu_sc as plsc

N = 16384
L = 16
NC = 2
NS = 16
NW = NC * NS
CHUNK = N // NW


def _lane_gather(vec, idx):
    return jnp.take_along_axis(vec, idx, axis=0, mode="promise_in_bounds")


def _sc_body(src_hbm, tgt_hbm, embT_hbm, w_hbm, b_hbm, out_hbm,
             src_v, tgt_v, out_v, embT_v, w_v, b_v):
    wid = lax.axis_index("s") * NC + lax.axis_index("c")
    base = wid * CHUNK

    pltpu.sync_copy(src_hbm.at[pl.ds(base, CHUNK)], src_v)
    pltpu.sync_copy(tgt_hbm.at[pl.ds(base, CHUNK)], tgt_v)
    pltpu.sync_copy(embT_hbm, embT_v)
    pltpu.sync_copy(w_hbm, w_v)
    pltpu.sync_copy(b_hbm, b_v)

    w_reg = w_v[...]
    v_acc = jnp.zeros((L,), jnp.float32)
    for j in range(L):
        wj = _lane_gather(w_reg, jnp.full((L,), j, jnp.int32))
        v_acc = v_acc + embT_v[j] * wj
    b_vec = b_v[...]

    for i in range(CHUNK // L):
        s_idx = src_v[pl.ds(i * L, L)]
        t_idx = tgt_v[pl.ds(i * L, L)]
        vs = _lane_gather(v_acc, s_idx)
        vt = _lane_gather(v_acc, t_idx)
        out_v[pl.ds(i * L, L)] = vs + vt + b_vec

    pltpu.sync_copy(out_v, out_hbm.at[pl.ds(base, CHUNK)])


def kernel(source, target, emb, W, b):
    embT_pad = jnp.zeros((L, L), jnp.float32).at[:, : emb.shape[0]].set(emb.T)
    w_flat = W.reshape(L)
    b_pad = jnp.broadcast_to(b.astype(jnp.float32), (L,))

    mesh = plsc.VectorSubcoreMesh(core_axis_name="c", subcore_axis_name="s")
    k = functools.partial(
        pl.kernel,
        mesh=mesh,
        out_type=jax.ShapeDtypeStruct((N,), jnp.float32),
        compiler_params=pltpu.CompilerParams(needs_layout_passes=False),
        scratch_types=[
            pltpu.VMEM((CHUNK,), jnp.int32),
            pltpu.VMEM((CHUNK,), jnp.int32),
            pltpu.VMEM((CHUNK,), jnp.float32),
            pltpu.VMEM((L, L), jnp.float32),
            pltpu.VMEM((L,), jnp.float32),
            pltpu.VMEM((L,), jnp.float32),
        ],
    )(_sc_body)
    out = k(source.astype(jnp.int32), target.astype(jnp.int32),
            embT_pad, w_flat, b_pad)
    return out.reshape(N, 1)


# speedup vs baseline: 5.3359x; 1.0697x over previous
import functools

import jax
import jax.numpy as jnp
from jax import lax
from jax.experimental import pallas as pl
from jax.experimental.pallas import tpu as pltpu
from jax.experimental.pallas import tpu_sc as plsc

N = 16384
L = 16
NC = 2
NS = 16
NW = NC * NS
CHUNK = N // NW


def _lane_gather(vec, idx):
    return jnp.take_along_axis(vec, idx, axis=0, mode="promise_in_bounds")


def _sc_body(src_hbm, tgt_hbm, pk_hbm, out_hbm,
             src_v, tgt_v, out_v, pk_v, sem):
    wid = lax.axis_index("s") * NC + lax.axis_index("c")
    base = wid * CHUNK

    c1 = pltpu.async_copy(pk_hbm, pk_v, sem)
    c2 = pltpu.async_copy(src_hbm.at[pl.ds(base, CHUNK)], src_v, sem)
    c3 = pltpu.async_copy(tgt_hbm.at[pl.ds(base, CHUNK)], tgt_v, sem)
    c1.wait()
    c2.wait()
    c3.wait()

    w_reg = pk_v[L]
    v_acc = jnp.zeros((L,), jnp.float32)
    for j in range(L):
        wj = _lane_gather(w_reg, jnp.full((L,), j, jnp.int32))
        v_acc = v_acc + pk_v[j] * wj
    b_vec = pk_v[L + 1]

    for i in range(CHUNK // L):
        s_idx = src_v[pl.ds(i * L, L)]
        t_idx = tgt_v[pl.ds(i * L, L)]
        vs = _lane_gather(v_acc, s_idx)
        vt = _lane_gather(v_acc, t_idx)
        out_v[pl.ds(i * L, L)] = vs + vt + b_vec

    pltpu.sync_copy(out_v, out_hbm.at[pl.ds(base, CHUNK)])


def kernel(source, target, emb, W, b):
    embT_pad = jnp.zeros((L, L), jnp.float32).at[:, : emb.shape[0]].set(emb.T)
    packed = jnp.concatenate(
        [embT_pad, W.reshape(1, L),
         jnp.broadcast_to(b.astype(jnp.float32), (1, L))], axis=0)

    mesh = plsc.VectorSubcoreMesh(core_axis_name="c", subcore_axis_name="s")
    k = functools.partial(
        pl.kernel,
        mesh=mesh,
        out_type=jax.ShapeDtypeStruct((N,), jnp.float32),
        compiler_params=pltpu.CompilerParams(needs_layout_passes=False),
        scratch_types=[
            pltpu.VMEM((CHUNK,), jnp.int32),
            pltpu.VMEM((CHUNK,), jnp.int32),
            pltpu.VMEM((CHUNK,), jnp.float32),
            pltpu.VMEM((L + 2, L), jnp.float32),
            pltpu.SemaphoreType.DMA,
        ],
    )(_sc_body)
    out = k(source.astype(jnp.int32), target.astype(jnp.int32), packed)
    return out.reshape(N, 1)


# speedup vs baseline: 5.8497x; 1.0963x over previous
import functools

import jax
import jax.numpy as jnp
from jax import lax
from jax.experimental import pallas as pl
from jax.experimental.pallas import tpu as pltpu
from jax.experimental.pallas import tpu_sc as plsc

N = 16384
L = 16
NC = 1
NS = 16
NW = NC * NS
CHUNK = N // NW


def _lane_gather(vec, idx):
    return jnp.take_along_axis(vec, idx, axis=0, mode="promise_in_bounds")


def _sc_body(src_hbm, tgt_hbm, pk_hbm, out_hbm,
             src_v, tgt_v, out_v, pk_v, sem):
    wid = lax.axis_index("s") * NC + lax.axis_index("c")
    base = wid * CHUNK

    c1 = pltpu.async_copy(pk_hbm, pk_v, sem)
    c2 = pltpu.async_copy(src_hbm.at[pl.ds(base, CHUNK)], src_v, sem)
    c3 = pltpu.async_copy(tgt_hbm.at[pl.ds(base, CHUNK)], tgt_v, sem)
    c1.wait()
    c2.wait()
    c3.wait()

    w_reg = pk_v[L]
    v_acc = jnp.zeros((L,), jnp.float32)
    for j in range(L):
        wj = _lane_gather(w_reg, jnp.full((L,), j, jnp.int32))
        v_acc = v_acc + pk_v[j] * wj
    b_vec = pk_v[L + 1]

    for i in range(CHUNK // L):
        s_idx = src_v[pl.ds(i * L, L)]
        t_idx = tgt_v[pl.ds(i * L, L)]
        vs = _lane_gather(v_acc, s_idx)
        vt = _lane_gather(v_acc, t_idx)
        out_v[pl.ds(i * L, L)] = vs + vt + b_vec

    pltpu.sync_copy(out_v, out_hbm.at[pl.ds(base, CHUNK)])


def kernel(source, target, emb, W, b):
    embT_pad = jnp.zeros((L, L), jnp.float32).at[:, : emb.shape[0]].set(emb.T)
    packed = jnp.concatenate(
        [embT_pad, W.reshape(1, L),
         jnp.broadcast_to(b.astype(jnp.float32), (1, L))], axis=0)

    mesh = plsc.VectorSubcoreMesh(
        core_axis_name="c", subcore_axis_name="s", num_cores=NC)
    k = functools.partial(
        pl.kernel,
        mesh=mesh,
        out_type=jax.ShapeDtypeStruct((N,), jnp.float32),
        compiler_params=pltpu.CompilerParams(needs_layout_passes=False),
        scratch_types=[
            pltpu.VMEM((CHUNK,), jnp.int32),
            pltpu.VMEM((CHUNK,), jnp.int32),
            pltpu.VMEM((CHUNK,), jnp.float32),
            pltpu.VMEM((L + 2, L), jnp.float32),
            pltpu.SemaphoreType.DMA,
        ],
    )(_sc_body)
    out = k(source.astype(jnp.int32), target.astype(jnp.int32), packed)
    return out.reshape(N, 1)
